# Initial kernel scaffold; baseline (speedup 1.0000x reference)
import jax
import jax.numpy as jnp
from jax.experimental import pallas as pl


def kernel(X, E, y, W_gcn, b_gcn, W_lin, b_lin, ln_gamma, ln_beta):
    raise NotImplementedError("write your pallas kernel here")


# speedup vs baseline: 581.1215x; 581.1215x over previous
import jax
import jax.numpy as jnp
from jax.experimental import pallas as pl
from jax.experimental.pallas import tpu as pltpu

_HI = jax.lax.Precision.HIGHEST


def _gnn_body(e_ref, x_ref, y_ref, wg_ref, bg_ref, wl_ref, bl_ref, g_ref,
              bt_ref, o_ref):
    n = e_ref.shape[1]
    hx = x_ref.shape[-1]

    ef = e_ref[0].astype(jnp.float32)
    k_i = jax.lax.broadcasted_iota(jnp.int32, (2 * n, n), 0)
    j_i = jax.lax.broadcasted_iota(jnp.int32, (2 * n, n), 1)
    sel_t = (k_i == 2 * j_i + 1).astype(jnp.float32)
    adj = jnp.dot(ef, sel_t)

    ones = jnp.ones((n, 1), jnp.float32)
    deg = jax.lax.dot_general(adj, ones, (((0,), (0,)), ((), ()))) + 1.0
    dis = jax.lax.rsqrt(deg)

    xw = jnp.dot(x_ref[0], wg_ref[...], precision=_HI)
    s = xw * dis
    agg = jax.lax.dot_general(adj, s, (((0,), (0,)), ((), ())),
                              precision=_HI)
    xa = dis * agg + (dis * dis) * xw + bg_ref[...]

    h = (jnp.dot(xa, wl_ref[:hx, :], precision=_HI)
         + jnp.dot(y_ref[0], wl_ref[hx:, :], precision=_HI)
         + bl_ref[...])
    h = jnp.maximum(h, 0.0)
    mu = jnp.mean(h, axis=1, keepdims=True)
    c = h - mu
    var = jnp.mean(c * c, axis=1, keepdims=True)
    hn = c * jax.lax.rsqrt(var + 1e-5)
    o_ref[0] = hn * g_ref[...] + bt_ref[...]


def kernel(X, E, y, W_gcn, b_gcn, W_lin, b_lin, ln_gamma, ln_beta):
    bs, n, hx = X.shape
    hy = y.shape[1]
    e2 = E.reshape(bs, n, 2 * n)
    y3 = y.reshape(bs, 1, hy)
    return pl.pallas_call(
        _gnn_body,
        grid=(bs,),
        in_specs=[
            pl.BlockSpec((1, n, 2 * n), lambda b: (b, 0, 0)),
            pl.BlockSpec((1, n, hx), lambda b: (b, 0, 0)),
            pl.BlockSpec((1, 1, hy), lambda b: (b, 0, 0)),
            pl.BlockSpec((hx, hx), lambda b: (0, 0)),
            pl.BlockSpec((1, hx), lambda b: (0, 0)),
            pl.BlockSpec((hx + hy, hx), lambda b: (0, 0)),
            pl.BlockSpec((1, hx), lambda b: (0, 0)),
            pl.BlockSpec((1, hx), lambda b: (0, 0)),
            pl.BlockSpec((1, hx), lambda b: (0, 0)),
        ],
        out_specs=pl.BlockSpec((1, n, hx), lambda b: (b, 0, 0)),
        out_shape=jax.ShapeDtypeStruct((bs, n, hx), X.dtype),
        compiler_params=pltpu.CompilerParams(
            dimension_semantics=("arbitrary",)),
    )(e2, X, y3, W_gcn, b_gcn.reshape(1, hx), W_lin, b_lin.reshape(1, hx),
      ln_gamma.reshape(1, hx), ln_beta.reshape(1, hx))


# speedup vs baseline: 1667.0850x; 2.8687x over previous
import jax
import jax.numpy as jnp
from jax.experimental import pallas as pl
from jax.experimental.pallas import tpu as pltpu

_HI = jax.lax.Precision.HIGHEST
_F32 = jnp.float32


def _split_dot_t(a_bf, v):
    v_hi = v.astype(jnp.bfloat16)
    v_lo = (v - v_hi.astype(_F32)).astype(jnp.bfloat16)
    dims = (((0,), (0,)), ((), ()))
    hi = jax.lax.dot_general(a_bf, v_hi, dims, preferred_element_type=_F32)
    lo = jax.lax.dot_general(a_bf, v_lo, dims, preferred_element_type=_F32)
    return hi + lo


def _gnn_body(a_ref, x_ref, y_ref, wg_ref, bg_ref, wl_ref, bl_ref, g_ref,
              bt_ref, o_ref):
    n = x_ref.shape[1]
    hx = x_ref.shape[-1]

    adj = a_ref[0]

    ones = jnp.ones((n, 1), jnp.bfloat16)
    deg = jax.lax.dot_general(adj, ones, (((0,), (0,)), ((), ())),
                              preferred_element_type=_F32) + 1.0
    dis = jax.lax.rsqrt(deg)

    xw = jnp.dot(x_ref[0], wg_ref[...], precision=_HI)
    agg = _split_dot_t(adj, xw * dis)
    xa = dis * agg + (dis * dis) * xw + bg_ref[...]

    h = (jnp.dot(xa, wl_ref[:hx, :], precision=_HI)
         + jnp.dot(y_ref[0], wl_ref[hx:, :], precision=_HI)
         + bl_ref[...])
    h = jnp.maximum(h, 0.0)
    mu = jnp.mean(h, axis=1, keepdims=True)
    c = h - mu
    var = jnp.mean(c * c, axis=1, keepdims=True)
    hn = c * jax.lax.rsqrt(var + 1e-5)
    o_ref[0] = hn * g_ref[...] + bt_ref[...]


def kernel(X, E, y, W_gcn, b_gcn, W_lin, b_lin, ln_gamma, ln_beta):
    bs, n, hx = X.shape
    hy = y.shape[1]
    adj = E[..., 1].astype(jnp.bfloat16)
    y3 = y.reshape(bs, 1, hy)
    return pl.pallas_call(
        _gnn_body,
        grid=(bs,),
        in_specs=[
            pl.BlockSpec((1, n, n), lambda b: (b, 0, 0)),
            pl.BlockSpec((1, n, hx), lambda b: (b, 0, 0)),
            pl.BlockSpec((1, 1, hy), lambda b: (b, 0, 0)),
            pl.BlockSpec((hx, hx), lambda b: (0, 0)),
            pl.BlockSpec((1, hx), lambda b: (0, 0)),
            pl.BlockSpec((hx + hy, hx), lambda b: (0, 0)),
            pl.BlockSpec((1, hx), lambda b: (0, 0)),
            pl.BlockSpec((1, hx), lambda b: (0, 0)),
            pl.BlockSpec((1, hx), lambda b: (0, 0)),
        ],
        out_specs=pl.BlockSpec((1, n, hx), lambda b: (b, 0, 0)),
        out_shape=jax.ShapeDtypeStruct((bs, n, hx), X.dtype),
        compiler_params=pltpu.CompilerParams(
            dimension_semantics=("arbitrary",)),
    )(adj, X, y3, W_gcn, b_gcn.reshape(1, hx), W_lin, b_lin.reshape(1, hx),
      ln_gamma.reshape(1, hx), ln_beta.reshape(1, hx))
